# baseline (speedup 1.0000x reference)
import functools

import jax
import jax.numpy as jnp
from jax.experimental import pallas as pl
from jax.experimental.pallas import tpu as pltpu

_FEAT = 30


def _leaky(x):
    return jnp.where(x >= 0, x, 0.01 * x)


def _fused_body(elems_ref, enc_ref, a1_ref, b1v_ref, bias1_ref,
                w2_ref, b2_ref, w3_ref, b3_ref, w4_ref, b4_ref,
                w5_ref, b5_ref, out_ref):
    elems = elems_ref[...]
    h = jnp.dot(elems, a1_ref[...], preferred_element_type=jnp.float32)
    h = h + jnp.dot(enc_ref[...], b1v_ref[...],
                    preferred_element_type=jnp.float32)
    h = _leaky(h + bias1_ref[...])
    h = _leaky(jnp.dot(h, w2_ref[...],
                       preferred_element_type=jnp.float32) + b2_ref[...])
    h = _leaky(jnp.dot(h, w3_ref[...],
                       preferred_element_type=jnp.float32) + b3_ref[...])
    h = _leaky(jnp.dot(h, w4_ref[...],
                       preferred_element_type=jnp.float32) + b4_ref[...])
    h = _leaky(jnp.dot(h, w5_ref[...],
                       preferred_element_type=jnp.float32) + b5_ref[...])
    xs = elems[:, _FEAT:_FEAT + 1]
    cort = h[:, 0:1]
    trab = h[:, 1:2]
    out = jnp.where(xs == 1.0, cort,
                    jnp.where(xs == 0.0, trab, jnp.zeros_like(cort)))
    out_ref[...] = jnp.maximum(out, 0.0)


def _block_diag_t(c, t):
    o, i = c.shape
    z = jnp.zeros((o, i), jnp.float32)
    top = jnp.concatenate([c, z], axis=1)
    bot = jnp.concatenate([z, t], axis=1)
    return jnp.concatenate([top, bot], axis=0).T


@functools.partial(jax.jit, static_argnames=())
def kernel(elems, encoded_features, cw1, cb1, cw2, cb2, cw3, cb3, cw4, cb4,
           cw5, cb5, tw1, tb1, tw2, tb2, tw3, tb3, tw4, tb4, tw5, tb5):
    b, e, f1 = elems.shape
    n = b * e
    cw = encoded_features.shape[-1]
    el = elems.reshape(n, f1)
    en = encoded_features.reshape(n, cw)

    w1t = jnp.concatenate([cw1, tw1], axis=0).T
    a1 = jnp.concatenate(
        [w1t[:_FEAT], jnp.zeros((f1 - _FEAT, 2 * cw1.shape[0]),
                                jnp.float32)], axis=0)
    b1v = w1t[_FEAT:]
    bias1 = jnp.concatenate([cb1, tb1])[None, :]

    w2 = _block_diag_t(cw2, tw2)
    b2 = jnp.concatenate([cb2, tb2])[None, :]
    w3 = _block_diag_t(cw3, tw3)
    b3 = jnp.concatenate([cb3, tb3])[None, :]
    w4 = _block_diag_t(cw4, tw4)
    b4 = jnp.concatenate([cb4, tb4])[None, :]
    w5 = _block_diag_t(cw5, tw5)
    b5 = jnp.concatenate([cb5, tb5])[None, :]

    blk = 2048
    grid = (n // blk,)
    full = lambda shape: pl.BlockSpec(shape, lambda i: (0, 0))
    out = pl.pallas_call(
        _fused_body,
        grid=grid,
        in_specs=[
            pl.BlockSpec((blk, f1), lambda i: (i, 0)),
            pl.BlockSpec((blk, cw), lambda i: (i, 0)),
            full(a1.shape),
            full(b1v.shape),
            full(bias1.shape),
            full(w2.shape),
            full(b2.shape),
            full(w3.shape),
            full(b3.shape),
            full(w4.shape),
            full(b4.shape),
            full(w5.shape),
            full(b5.shape),
        ],
        out_specs=pl.BlockSpec((blk, 1), lambda i: (i, 0)),
        out_shape=jax.ShapeDtypeStruct((n, 1), jnp.float32),
        compiler_params=pltpu.CompilerParams(
            dimension_semantics=("arbitrary",),
        ),
    )(el, en, a1, b1v, bias1, w2, b2, w3, b3, w4, b4, w5, b5)
    return out.reshape(b, e, 1)


# speedup vs baseline: 1.0022x; 1.0022x over previous
import functools

import jax
import jax.numpy as jnp
from jax.experimental import pallas as pl
from jax.experimental.pallas import tpu as pltpu

_FEAT = 30


def _leaky(x):
    return jnp.maximum(x, 0.01 * x)


def _fused_body(elems_ref, enc_ref, a1_ref, b1v_ref, bias1_ref,
                w2_ref, b2_ref, w3_ref, b3_ref, w4_ref, b4_ref,
                w5_ref, b5_ref, out_ref):
    elems = elems_ref[...]
    h = jnp.dot(elems, a1_ref[...], preferred_element_type=jnp.float32)
    h = h + jnp.dot(enc_ref[...], b1v_ref[...],
                    preferred_element_type=jnp.float32)
    h = _leaky(h + bias1_ref[...])
    h = _leaky(jnp.dot(h, w2_ref[...],
                       preferred_element_type=jnp.float32) + b2_ref[...])
    h = _leaky(jnp.dot(h, w3_ref[...],
                       preferred_element_type=jnp.float32) + b3_ref[...])
    h = _leaky(jnp.dot(h, w4_ref[...],
                       preferred_element_type=jnp.float32) + b4_ref[...])
    h = _leaky(jnp.dot(h, w5_ref[...],
                       preferred_element_type=jnp.float32) + b5_ref[...])
    xs = elems[:, _FEAT:_FEAT + 1]
    cort = h[:, 0:1]
    trab = h[:, 1:2]
    out = jnp.where(xs == 1.0, cort,
                    jnp.where(xs == 0.0, trab, jnp.zeros_like(cort)))
    out_ref[...] = jnp.maximum(out, 0.0)


def _block_diag_t(c, t):
    o, i = c.shape
    z = jnp.zeros((o, i), jnp.float32)
    top = jnp.concatenate([c, z], axis=1)
    bot = jnp.concatenate([z, t], axis=1)
    return jnp.concatenate([top, bot], axis=0).T


@functools.partial(jax.jit, static_argnames=())
def kernel(elems, encoded_features, cw1, cb1, cw2, cb2, cw3, cb3, cw4, cb4,
           cw5, cb5, tw1, tb1, tw2, tb2, tw3, tb3, tw4, tb4, tw5, tb5):
    b, e, f1 = elems.shape
    n = b * e
    cw = encoded_features.shape[-1]
    el = elems.reshape(n, f1)
    en = encoded_features.reshape(n, cw)

    w1t = jnp.concatenate([cw1, tw1], axis=0).T
    a1 = jnp.concatenate(
        [w1t[:_FEAT], jnp.zeros((f1 - _FEAT, 2 * cw1.shape[0]),
                                jnp.float32)], axis=0)
    b1v = w1t[_FEAT:]
    bias1 = jnp.concatenate([cb1, tb1])[None, :]

    w2 = _block_diag_t(cw2, tw2)
    b2 = jnp.concatenate([cb2, tb2])[None, :]
    w3 = _block_diag_t(cw3, tw3)
    b3 = jnp.concatenate([cb3, tb3])[None, :]
    w4 = _block_diag_t(cw4, tw4)
    b4 = jnp.concatenate([cb4, tb4])[None, :]
    w5 = _block_diag_t(cw5, tw5)
    b5 = jnp.concatenate([cb5, tb5])[None, :]

    blk = 2048
    grid = (n // blk,)
    full = lambda shape: pl.BlockSpec(shape, lambda i: (0, 0))
    out = pl.pallas_call(
        _fused_body,
        grid=grid,
        in_specs=[
            pl.BlockSpec((blk, f1), lambda i: (i, 0)),
            pl.BlockSpec((blk, cw), lambda i: (i, 0)),
            full(a1.shape),
            full(b1v.shape),
            full(bias1.shape),
            full(w2.shape),
            full(b2.shape),
            full(w3.shape),
            full(b3.shape),
            full(w4.shape),
            full(b4.shape),
            full(w5.shape),
            full(b5.shape),
        ],
        out_specs=pl.BlockSpec((blk, 1), lambda i: (i, 0)),
        out_shape=jax.ShapeDtypeStruct((n, 1), jnp.float32),
        compiler_params=pltpu.CompilerParams(
            dimension_semantics=("arbitrary",),
        ),
    )(el, en, a1, b1v, bias1, w2, b2, w3, b3, w4, b4, w5, b5)
    return out.reshape(b, e, 1)
